# baseline (speedup 1.0000x reference)
import functools

import jax
import jax.numpy as jnp
from jax import lax
from jax.experimental import pallas as pl
from jax.experimental.pallas import tpu as pltpu
from jax.experimental.pallas import tpu_sc as plsc

NC = 2
NS = 16
NW = NC * NS
L = 16

CB = 8
JP = 128


def _vreg_take(v, idx):
    return lax.gather(
        v, idx[:, None],
        lax.GatherDimensionNumbers(
            offset_dims=(), collapsed_slice_dims=(0,), start_index_map=(0,)),
        (1,),
        mode=lax.GatherScatterMode.PROMISE_IN_BOUNDS,
    )


def _sc_dots(labels, input_labels, out_table, in_table, B, J, H):
    bpw = B // NW
    mesh = plsc.VectorSubcoreMesh(core_axis_name="c", subcore_axis_name="s")
    n_grp = JP // L

    NR = CB * J + 8

    @functools.partial(
        pl.kernel,
        out_type=jax.ShapeDtypeStruct((B, JP), jnp.float32),
        mesh=mesh,
        compiler_params=pltpu.CompilerParams(
            use_tc_tiling_on_sc=False, needs_layout_passes=False),
        scratch_types=[
            pltpu.VMEM((2, CB, JP), jnp.int32),
            pltpu.VMEM((2, NR, H), jnp.float32),
            pltpu.VMEM((2, CB), jnp.int32),
            pltpu.VMEM((2, CB, H), jnp.float32),
            pltpu.VMEM((2, CB, JP), jnp.float32),
            pltpu.SemaphoreType.DMA,
            pltpu.SemaphoreType.DMA,
            pltpu.SemaphoreType.DMA,
            pltpu.SemaphoreType.DMA,
            pltpu.SemaphoreType.DMA,
            pltpu.SemaphoreType.DMA,
            pltpu.SemaphoreType.DMA,
            pltpu.SemaphoreType.DMA,
        ],
    )
    def dots_kernel(labels_hbm, inlab_hbm, outtab_hbm, intab_hbm,
                    dots_hbm,
                    idx_v, rows_v, idx2_v, inrows_v, dots_v,
                    semr0, semr1, semi0, semi1, semx0, semx1, semd0, semd1):
        wid = lax.axis_index("s") * NC + lax.axis_index("c")
        base_b = wid * bpw
        semr = [semr0, semr1]
        semi = [semi0, semi1]
        semx = [semx0, semx1]
        semd = [semd0, semd1]

        def idx_copies(s, b0, make=False):
            ctor = pltpu.make_async_copy if make else pltpu.async_copy
            return [
                ctor(labels_hbm.at[pl.ds(b0, CB)], idx_v.at[s], semx[s]),
                ctor(inlab_hbm.at[pl.ds(b0, CB)], idx2_v.at[s], semx[s]),
            ]

        def gather_copies(s, make=False):
            ctor = pltpu.make_async_copy if make else pltpu.async_copy
            cps = [
                ctor(outtab_hbm.at[idx_v.at[s].at[c].at[pl.ds(0, J)]],
                     rows_v.at[s].at[pl.ds(c * J, J)], semr[s])
                for c in range(CB)
            ]
            cps.append(ctor(intab_hbm.at[idx2_v.at[s]], inrows_v.at[s], semi[s]))
            return cps

        def compute(s, b0):
            rv = rows_v.at[s]
            dv = dots_v.at[s]
            iota = lax.iota(jnp.int32, L)

            @pl.loop(0, CB)
            def _(c):
                jvecs = [c * J + g * L + iota for g in range(n_grp)]
                evs = [inrows_v[s, c, pl.ds(k * L, L)] for k in range(H // L)]

                def mk_hbody(k):
                    ea, eb = evs[k], evs[(k + 1) % (H // L)]

                    def hbody(hh, carry):
                        accs, hvec = carry
                        widx = (hh + iota) & (L - 1)
                        rota = _vreg_take(ea, widx)
                        rotb = _vreg_take(eb, widx)
                        ew = jnp.where(iota < L - hh, rota, rotb)
                        new = tuple(
                            accs[g] + ew * plsc.load_gather(rv, [jvecs[g], hvec])
                            for g in range(n_grp)
                        )
                        return new, (hvec + 1) & (H - 1)

                    return hbody

                carry = (tuple(jnp.zeros((L,), jnp.float32)
                               for _ in range(n_grp)),
                         iota)
                for k in range(H // L):
                    carry = lax.fori_loop(0, L, mk_hbody(k), carry, unroll=4)
                accs = carry[0]
                for g in range(n_grp):
                    dv[c, pl.ds(g * L, L)] = accs[g]

            pltpu.async_copy(dv, dots_hbm.at[pl.ds(b0, CB)], semd[s])

        def dots_wait(s, b0):
            pltpu.make_async_copy(
                dots_v.at[s], dots_hbm.at[pl.ds(b0, CB)], semd[s]).wait()

        def half(s, b0, nxt, first):
            for c in gather_copies(s, make=True):
                c.wait()
            idx_copies(s, nxt)
            if not first:
                dots_wait(s, b0)
            compute(s, b0)
            for c in idx_copies(s, nxt, make=True):
                c.wait()
            gather_copies(s)

        for s in range(2):
            for c in idx_copies(s, base_b + s * CB):
                c.wait()
            gather_copies(s)

        for s in range(2):
            half(s, base_b + s * CB, base_b + s * CB + 2 * CB, first=True)

        @pl.loop(2 * CB, bpw, step=2 * CB)
        def _(cb0):
            for s in range(2):
                b0 = base_b + cb0 + s * CB
                nxt = base_b + lax.rem(cb0 + s * CB + 2 * CB, bpw)
                half(s, b0, nxt, first=False)

        for s in range(2):
            for c in gather_copies(s, make=True):
                c.wait()
            dots_wait(s, base_b + s * CB)

    return dots_kernel(labels, input_labels, out_table, in_table)


def _tc_loss(dots, B, P, J):
    BB = 4096

    def body(d_ref, o_ref):
        d = d_ref[...]
        j = lax.broadcasted_iota(jnp.int32, (BB, JP), 1)
        x = jnp.where(j < P, d, -d) + 1e-9
        ls = jnp.minimum(x, 0.0) - jnp.log1p(jnp.exp(-jnp.abs(x)))
        ls = jnp.where(j < J, ls, 0.0)
        o_ref[...] = -jnp.sum(ls, axis=1)

    return pl.pallas_call(
        body,
        grid=(B // BB,),
        in_specs=[pl.BlockSpec((BB, JP), lambda i: (i, 0))],
        out_specs=pl.BlockSpec((BB,), lambda i: (i,)),
        out_shape=jax.ShapeDtypeStruct((B,), jnp.float32),
    )(dots)


def kernel(input_labels, pos_labels, neg_labels, in_table, out_table):
    B = input_labels.shape[0]
    P = pos_labels.shape[1]
    N = neg_labels.shape[1]
    H = in_table.shape[1]
    J = P + N

    labels = jnp.concatenate(
        [pos_labels, neg_labels, jnp.zeros((B, JP - J), jnp.int32)],
        axis=1)
    dots = _sc_dots(labels, input_labels, out_table, in_table, B, J, H)
    return _tc_loss(dots, B, P, J)


# speedup vs baseline: 1.1413x; 1.1413x over previous
import functools

import jax
import jax.numpy as jnp
from jax import lax
from jax.experimental import pallas as pl
from jax.experimental.pallas import tpu as pltpu
from jax.experimental.pallas import tpu_sc as plsc

NC = 2
NS = 16
NW = NC * NS
L = 16

CB = 8
JP = 128


def _sc_dots(labels, input_labels, out_table, in_table, B, J, H):
    bpw = B // NW
    mesh = plsc.VectorSubcoreMesh(core_axis_name="c", subcore_axis_name="s")
    n_grp = JP // L

    NR = CB * J + 8

    @functools.partial(
        pl.kernel,
        out_type=jax.ShapeDtypeStruct((B, JP), jnp.float32),
        mesh=mesh,
        compiler_params=pltpu.CompilerParams(
            use_tc_tiling_on_sc=False, needs_layout_passes=False),
        scratch_types=[
            pltpu.VMEM((2, CB, JP), jnp.int32),
            pltpu.VMEM((2, NR, H), jnp.float32),
            pltpu.VMEM((2, CB), jnp.int32),
            pltpu.VMEM((2, CB, H), jnp.float32),
            pltpu.VMEM((CB, H + L), jnp.float32),
            pltpu.VMEM((2, CB, JP), jnp.float32),
            pltpu.SemaphoreType.DMA,
            pltpu.SemaphoreType.DMA,
            pltpu.SemaphoreType.DMA,
            pltpu.SemaphoreType.DMA,
            pltpu.SemaphoreType.DMA,
            pltpu.SemaphoreType.DMA,
            pltpu.SemaphoreType.DMA,
            pltpu.SemaphoreType.DMA,
        ],
    )
    def dots_kernel(labels_hbm, inlab_hbm, outtab_hbm, intab_hbm,
                    dots_hbm,
                    idx_v, rows_v, idx2_v, inrows_v, edup_v, dots_v,
                    semr0, semr1, semi0, semi1, semx0, semx1, semd0, semd1):
        wid = lax.axis_index("s") * NC + lax.axis_index("c")
        base_b = wid * bpw
        semr = [semr0, semr1]
        semi = [semi0, semi1]
        semx = [semx0, semx1]
        semd = [semd0, semd1]

        def idx_copies(s, b0, make=False):
            ctor = pltpu.make_async_copy if make else pltpu.async_copy
            return [
                ctor(labels_hbm.at[pl.ds(b0, CB)], idx_v.at[s], semx[s]),
                ctor(inlab_hbm.at[pl.ds(b0, CB)], idx2_v.at[s], semx[s]),
            ]

        def gather_copies(s, make=False):
            ctor = pltpu.make_async_copy if make else pltpu.async_copy
            cps = [
                ctor(outtab_hbm.at[idx_v.at[s].at[c].at[pl.ds(0, J)]],
                     rows_v.at[s].at[pl.ds(c * J, J)], semr[s])
                for c in range(CB)
            ]
            cps.append(ctor(intab_hbm.at[idx2_v.at[s]], inrows_v.at[s], semi[s]))
            return cps

        def compute(s, b0):
            for c in range(CB):
                for k in range(H // L):
                    edup_v[c, pl.ds(k * L, L)] = inrows_v[s, c, pl.ds(k * L, L)]
                edup_v[c, pl.ds(H, L)] = inrows_v[s, c, pl.ds(0, L)]

            rv = rows_v.at[s]
            dv = dots_v.at[s]

            @pl.loop(0, CB)
            def _(c):
                jvecs = [c * J + g * L + lax.iota(jnp.int32, L)
                         for g in range(n_grp)]

                def hbody(h, carry):
                    accs, hvec = carry
                    ew = edup_v[c, pl.ds(h, L)]
                    new = tuple(
                        accs[g] + ew * plsc.load_gather(rv, [jvecs[g], hvec])
                        for g in range(n_grp)
                    )
                    return new, (hvec + 1) & (H - 1)

                accs, _ = lax.fori_loop(
                    0, H, hbody,
                    (tuple(jnp.zeros((L,), jnp.float32) for _ in range(n_grp)),
                     lax.iota(jnp.int32, L)),
                    unroll=4)
                for g in range(n_grp):
                    dv[c, pl.ds(g * L, L)] = accs[g]

            pltpu.async_copy(dv, dots_hbm.at[pl.ds(b0, CB)], semd[s])

        def dots_wait(s, b0):
            pltpu.make_async_copy(
                dots_v.at[s], dots_hbm.at[pl.ds(b0, CB)], semd[s]).wait()

        def half(s, b0, nxt, first):
            for c in gather_copies(s, make=True):
                c.wait()
            idx_copies(s, nxt)
            if not first:
                dots_wait(s, b0)
            compute(s, b0)
            for c in idx_copies(s, nxt, make=True):
                c.wait()
            gather_copies(s)

        for s in range(2):
            for c in idx_copies(s, base_b + s * CB):
                c.wait()
            gather_copies(s)

        for s in range(2):
            half(s, base_b + s * CB, base_b + s * CB + 2 * CB, first=True)

        @pl.loop(2 * CB, bpw, step=2 * CB)
        def _(cb0):
            for s in range(2):
                b0 = base_b + cb0 + s * CB
                nxt = base_b + lax.rem(cb0 + s * CB + 2 * CB, bpw)
                half(s, b0, nxt, first=False)

        for s in range(2):
            for c in gather_copies(s, make=True):
                c.wait()
            dots_wait(s, base_b + s * CB)

    return dots_kernel(labels, input_labels, out_table, in_table)


def _tc_loss(dots, B, P, J):
    BB = 4096

    def body(d_ref, o_ref):
        d = d_ref[...]
        j = lax.broadcasted_iota(jnp.int32, (BB, JP), 1)
        x = jnp.where(j < P, d, -d) + 1e-9
        ls = jnp.minimum(x, 0.0) - jnp.log1p(jnp.exp(-jnp.abs(x)))
        ls = jnp.where(j < J, ls, 0.0)
        o_ref[...] = -jnp.sum(ls, axis=1)

    return pl.pallas_call(
        body,
        grid=(B // BB,),
        in_specs=[pl.BlockSpec((BB, JP), lambda i: (i, 0))],
        out_specs=pl.BlockSpec((BB,), lambda i: (i,)),
        out_shape=jax.ShapeDtypeStruct((B,), jnp.float32),
    )(dots)


def kernel(input_labels, pos_labels, neg_labels, in_table, out_table):
    B = input_labels.shape[0]
    P = pos_labels.shape[1]
    N = neg_labels.shape[1]
    H = in_table.shape[1]
    J = P + N

    labels = jnp.concatenate(
        [pos_labels, neg_labels, jnp.zeros((B, JP - J), jnp.int32)],
        axis=1)
    dots = _sc_dots(labels, input_labels, out_table, in_table, B, J, H)
    return _tc_loss(dots, B, P, J)
